# Initial kernel scaffold; baseline (speedup 1.0000x reference)
import jax
import jax.numpy as jnp
from jax.experimental import pallas as pl


def kernel(x, W_rel, b_rel, W_root, edge_index, batch):
    raise NotImplementedError("write your pallas kernel here")


# speedup vs baseline: 36.3725x; 36.3725x over previous
import functools

import jax
import jax.numpy as jnp
from jax import lax
from jax.experimental import pallas as pl
from jax.experimental.pallas import tpu as pltpu
from jax.experimental.pallas import tpu_sc as plsc

N = 10000
E = 320000
D = 128
G = 64
NC = 2
NS = 16
NW = NC * NS
EPW = E // NW
L = 16


def _tc1_body(x_ref, wr_ref, wo_ref, b_ref, srel_ref, sroot_ref):
    x = x_ref[...]
    srel_ref[...] = jnp.sum(x * wr_ref[...], axis=1)
    sroot_ref[...] = jnp.sum(x * wo_ref[...], axis=1) + b_ref[0, 0]


def _tc1(x, w_rel, w_root, b_rel):
    return pl.pallas_call(
        _tc1_body,
        out_shape=[
            jax.ShapeDtypeStruct((N,), jnp.float32),
            jax.ShapeDtypeStruct((N,), jnp.float32),
        ],
    )(x, w_rel, w_root, b_rel)


def _sc_scatter_body(edge_hbm, srel_hbm, out_hbm, src_v, dst_v, srel_v, acc_v):
    wid = lax.axis_index("s") * NC + lax.axis_index("c")
    base = wid * EPW
    pltpu.sync_copy(edge_hbm.at[pl.ds(base, EPW)], src_v)
    pltpu.sync_copy(edge_hbm.at[pl.ds(E + base, EPW)], dst_v)
    pltpu.sync_copy(srel_hbm, srel_v)

    def _zero(i, carry):
        acc_v[pl.ds(i * L, L)] = jnp.zeros((L,), jnp.float32)
        return carry

    lax.fori_loop(0, N // L, _zero, 0)

    def _edge(i, carry):
        s = src_v[pl.ds(i * L, L)]
        d = dst_v[pl.ds(i * L, L)]
        v = plsc.load_gather(srel_v, [s])
        plsc.addupdate_scatter(acc_v, [d], v)
        return carry

    lax.fori_loop(0, EPW // L, _edge, 0)

    pltpu.sync_copy(acc_v, out_hbm.at[wid])


@functools.cache
def _sc_scatter():
    mesh = plsc.VectorSubcoreMesh(core_axis_name="c", subcore_axis_name="s",
                                  num_cores=NC, num_subcores=NS)
    return pl.kernel(
        _sc_scatter_body,
        out_type=jax.ShapeDtypeStruct((NW, N), jnp.float32),
        mesh=mesh,
        compiler_params=pltpu.CompilerParams(needs_layout_passes=False),
        scratch_types=[
            pltpu.VMEM((EPW,), jnp.int32),
            pltpu.VMEM((EPW,), jnp.int32),
            pltpu.VMEM((N,), jnp.float32),
            pltpu.VMEM((N,), jnp.float32),
        ],
    )


def _tc2_body(x_ref, part_ref, sroot_ref, batch_ref, out_ref):
    xconv = jnp.sum(part_ref[...], axis=0) + sroot_ref[...]
    b = batch_ref[...]
    seg = lax.broadcasted_iota(jnp.int32, (G, N), 0)
    mask = seg == b[None, :]
    neg = jnp.float32(-jnp.inf)
    xb = jnp.where(mask, xconv[None, :], neg)
    m = jnp.max(xb, axis=1)
    m = jnp.where(jnp.isfinite(m), m, 0.0)
    t = jnp.where(mask, xconv[None, :] - m[:, None], neg)
    e = jnp.exp(t)
    denom = jnp.sum(e, axis=1)
    scores = e / (denom[:, None] + 1e-16)
    out_ref[...] = jnp.dot(scores, x_ref[...],
                           preferred_element_type=jnp.float32,
                           precision=lax.Precision.HIGHEST)


def _tc2(x, part, sroot, batch):
    return pl.pallas_call(
        _tc2_body,
        out_shape=jax.ShapeDtypeStruct((G, D), jnp.float32),
    )(x, part, sroot, batch)


def kernel(x, W_rel, b_rel, W_root, edge_index, batch):
    srel, sroot = _tc1(x, W_rel.reshape(1, D), W_root.reshape(1, D),
                       b_rel.reshape(1, 1))
    part = _sc_scatter()(edge_index.reshape(2 * E), srel)
    return _tc2(x, part, sroot, batch)


# speedup vs baseline: 38.3127x; 1.0533x over previous
import functools

import jax
import jax.numpy as jnp
from jax import lax
from jax.experimental import pallas as pl
from jax.experimental.pallas import tpu as pltpu
from jax.experimental.pallas import tpu_sc as plsc

N = 10000
E = 320000
D = 128
G = 64
NC = 2
NS = 16
NW = NC * NS
L = 16

EPW0 = 9984
TAIL_BASE = NW * EPW0
TAIL_PER_TILE = 128


def _tc1_body(x_ref, w2_ref, b_ref, srel_ref, sroot_ref):
    s2 = lax.dot_general(x_ref[...], w2_ref[...], (((1,), (0,)), ((), ())),
                         preferred_element_type=jnp.float32,
                         precision=lax.Precision.HIGHEST)
    srel_ref[...] = s2[:, 0]
    sroot_ref[...] = s2[:, 1] + b_ref[0, 0]


def _tc1(x, w2, b_rel):
    return pl.pallas_call(
        _tc1_body,
        out_shape=[
            jax.ShapeDtypeStruct((N,), jnp.float32),
            jax.ShapeDtypeStruct((N,), jnp.float32),
        ],
    )(x, w2, b_rel)


def _sc_scatter_body(edge_hbm, srel_hbm, out_hbm, eslab_v, etail_v, srel_v,
                     acc_v, sem_e, sem_t, sem_s):
    wid = lax.axis_index("s") * NC + lax.axis_index("c")

    cp_e = pltpu.async_copy(edge_hbm.at[:, pl.ds(wid * EPW0, EPW0)],
                            eslab_v, sem_e)
    cp_s = pltpu.async_copy(srel_hbm, srel_v, sem_s)

    @pl.when(wid < 4)
    def _():
        pltpu.async_copy(
            edge_hbm.at[:, pl.ds(TAIL_BASE + wid * TAIL_PER_TILE,
                                 TAIL_PER_TILE)], etail_v, sem_t)

    def _zero(i, carry):
        acc_v[pl.ds(i * L, L)] = jnp.zeros((L,), jnp.float32)
        return carry

    lax.fori_loop(0, N // L, _zero, 0)

    cp_s.wait()
    cp_e.wait()

    def _edges(i, carry):
        s = eslab_v[0, pl.ds(i * L, L)]
        d = eslab_v[1, pl.ds(i * L, L)]
        v = plsc.load_gather(srel_v, [s])
        plsc.addupdate_scatter(acc_v, [d], v)
        return carry

    lax.fori_loop(0, EPW0 // L, _edges, 0, unroll=4)

    @pl.when(wid < 4)
    def _():
        pltpu.make_async_copy(
            edge_hbm.at[:, pl.ds(TAIL_BASE, TAIL_PER_TILE)],
            etail_v, sem_t).wait()

        def _tail(i, carry):
            s = etail_v[0, pl.ds(i * L, L)]
            d = etail_v[1, pl.ds(i * L, L)]
            v = plsc.load_gather(srel_v, [s])
            plsc.addupdate_scatter(acc_v, [d], v)
            return carry

        lax.fori_loop(0, TAIL_PER_TILE // L, _tail, 0, unroll=4)

    pltpu.sync_copy(acc_v, out_hbm.at[wid])


@functools.cache
def _sc_scatter():
    mesh = plsc.VectorSubcoreMesh(core_axis_name="c", subcore_axis_name="s",
                                  num_cores=NC, num_subcores=NS)
    return pl.kernel(
        _sc_scatter_body,
        out_type=jax.ShapeDtypeStruct((NW, N), jnp.float32),
        mesh=mesh,
        compiler_params=pltpu.CompilerParams(needs_layout_passes=False),
        scratch_types=[
            pltpu.VMEM((2, EPW0), jnp.int32),
            pltpu.VMEM((2, TAIL_PER_TILE), jnp.int32),
            pltpu.VMEM((N,), jnp.float32),
            pltpu.VMEM((N,), jnp.float32),
            pltpu.SemaphoreType.DMA,
            pltpu.SemaphoreType.DMA,
            pltpu.SemaphoreType.DMA,
        ],
    )


def _tc2_body(x_ref, part_ref, sroot_ref, batch_ref, out_ref):
    xconv = jnp.sum(part_ref[...], axis=0) + sroot_ref[...]
    b = batch_ref[...]
    seg = lax.broadcasted_iota(jnp.int32, (G, N), 0)
    mask = seg == b[None, :]
    neg = jnp.float32(-jnp.inf)
    xb = jnp.where(mask, xconv[None, :], neg)
    m = jnp.max(xb, axis=1)
    m = jnp.where(jnp.isfinite(m), m, 0.0)
    t = jnp.where(mask, xconv[None, :] - m[:, None], neg)
    e = jnp.exp(t)
    denom = jnp.sum(e, axis=1)
    scores = e / (denom[:, None] + 1e-16)
    out_ref[...] = jnp.dot(scores, x_ref[...],
                           preferred_element_type=jnp.float32,
                           precision=lax.Precision.HIGHEST)


def _tc2(x, part, sroot, batch):
    return pl.pallas_call(
        _tc2_body,
        out_shape=jax.ShapeDtypeStruct((G, D), jnp.float32),
    )(x, part, sroot, batch)


def kernel(x, W_rel, b_rel, W_root, edge_index, batch):
    w2 = jnp.concatenate([W_rel, W_root], axis=1)
    srel, sroot = _tc1(x, w2, b_rel.reshape(1, 1))
    part = _sc_scatter()(edge_index, srel)
    return _tc2(x, part, sroot, batch)


# speedup vs baseline: 48.7037x; 1.2712x over previous
import functools

import jax
import jax.numpy as jnp
from jax import lax
from jax.experimental import pallas as pl
from jax.experimental.pallas import tpu as pltpu
from jax.experimental.pallas import tpu_sc as plsc

N = 10000
E = 320000
D = 128
G = 64
NC = 2
NS = 16
NW = NC * NS
L = 16

EPW0 = 9984
TAIL_BASE = NW * EPW0
TAIL_PER_TILE = 128


def _tc1_body(x_ref, w2t_ref, b_ref, srel_ref, sroot_ref):
    s2t = lax.dot_general(w2t_ref[...], x_ref[...], (((1,), (1,)), ((), ())),
                          preferred_element_type=jnp.float32,
                          precision=lax.Precision.HIGHEST)
    srel_ref[...] = s2t[0, :]
    sroot_ref[...] = s2t[1, :] + b_ref[0, 0]


def _tc1(x, w2, b_rel):
    return pl.pallas_call(
        _tc1_body,
        out_shape=[
            jax.ShapeDtypeStruct((N,), jnp.float32),
            jax.ShapeDtypeStruct((N,), jnp.float32),
        ],
    )(x, w2, b_rel)


def _sc_scatter_body(edge_hbm, srel_hbm, out_hbm, eslab_v, etail_v, srel_v,
                     acc_v, sem_e, sem_t, sem_s):
    wid = lax.axis_index("s") * NC + lax.axis_index("c")

    cp_e = pltpu.async_copy(edge_hbm.at[:, pl.ds(wid * EPW0, EPW0)],
                            eslab_v, sem_e)
    cp_s = pltpu.async_copy(srel_hbm, srel_v, sem_s)

    @pl.when(wid < 4)
    def _():
        pltpu.async_copy(
            edge_hbm.at[:, pl.ds(TAIL_BASE + wid * TAIL_PER_TILE,
                                 TAIL_PER_TILE)], etail_v, sem_t)

    def _zero(i, carry):
        acc_v[pl.ds(i * L, L)] = jnp.zeros((L,), jnp.float32)
        return carry

    lax.fori_loop(0, N // L, _zero, 0)

    cp_s.wait()
    cp_e.wait()

    def _edges(i, carry):
        s = eslab_v[0, pl.ds(i * L, L)]
        d = eslab_v[1, pl.ds(i * L, L)]
        v = plsc.load_gather(srel_v, [s])
        plsc.addupdate_scatter(acc_v, [d], v)
        return carry

    lax.fori_loop(0, EPW0 // L, _edges, 0, unroll=4)

    @pl.when(wid < 4)
    def _():
        pltpu.make_async_copy(
            edge_hbm.at[:, pl.ds(TAIL_BASE, TAIL_PER_TILE)],
            etail_v, sem_t).wait()

        def _tail(i, carry):
            s = etail_v[0, pl.ds(i * L, L)]
            d = etail_v[1, pl.ds(i * L, L)]
            v = plsc.load_gather(srel_v, [s])
            plsc.addupdate_scatter(acc_v, [d], v)
            return carry

        lax.fori_loop(0, TAIL_PER_TILE // L, _tail, 0, unroll=4)

    pltpu.sync_copy(acc_v, out_hbm.at[wid])


@functools.cache
def _sc_scatter():
    mesh = plsc.VectorSubcoreMesh(core_axis_name="c", subcore_axis_name="s",
                                  num_cores=NC, num_subcores=NS)
    return pl.kernel(
        _sc_scatter_body,
        out_type=jax.ShapeDtypeStruct((NW, N), jnp.float32),
        mesh=mesh,
        compiler_params=pltpu.CompilerParams(needs_layout_passes=False),
        scratch_types=[
            pltpu.VMEM((2, EPW0), jnp.int32),
            pltpu.VMEM((2, TAIL_PER_TILE), jnp.int32),
            pltpu.VMEM((N,), jnp.float32),
            pltpu.VMEM((N,), jnp.float32),
            pltpu.SemaphoreType.DMA,
            pltpu.SemaphoreType.DMA,
            pltpu.SemaphoreType.DMA,
        ],
    )


def _tc2_body(x_ref, part_ref, sroot_ref, batch_ref, out_ref):
    xconv = jnp.sum(part_ref[...], axis=0) + sroot_ref[...]
    b = batch_ref[...]
    seg = lax.broadcasted_iota(jnp.int32, (G, N), 0)
    mask = seg == b[None, :]
    neg = jnp.float32(-jnp.inf)
    xb = jnp.where(mask, xconv[None, :], neg)
    m = jnp.max(xb, axis=1)
    m = jnp.where(jnp.isfinite(m), m, 0.0)
    t = jnp.where(mask, xconv[None, :] - m[:, None], neg)
    e = jnp.exp(t)
    denom = jnp.sum(e, axis=1)
    scores = e / (denom[:, None] + 1e-16)
    out_ref[...] = jnp.dot(scores, x_ref[...],
                           preferred_element_type=jnp.float32)


def _tc2(x, part, sroot, batch):
    return pl.pallas_call(
        _tc2_body,
        out_shape=jax.ShapeDtypeStruct((G, D), jnp.float32),
    )(x, part, sroot, batch)


def kernel(x, W_rel, b_rel, W_root, edge_index, batch):
    w2t = jnp.concatenate([W_rel.reshape(1, D), W_root.reshape(1, D)], axis=0)
    srel, sroot = _tc1(x, w2t, b_rel.reshape(1, 1))
    part = _sc_scatter()(edge_index, srel)
    return _tc2(x, part, sroot, batch)
